# baseline (speedup 1.0000x reference)
import functools

import jax
import jax.numpy as jnp
from jax import lax
from jax.experimental import pallas as pl
from jax.experimental.pallas import tpu as pltpu
from jax.experimental.pallas import tpu_sc as plsc

_LANES = 16
_IDXW = 100


@functools.lru_cache(maxsize=None)
def _build_embed(rows, emb, seq):
    info = plsc.get_sparse_core_info()
    nc, ns = info.num_cores, info.num_subcores
    nw = nc * ns
    assert rows % nw == 0
    rpw = rows // nw
    chunk = 4 * seq
    assert rpw % chunk == 0 and chunk % _IDXW == 0
    nch = rpw // chunk
    ng = chunk // _IDXW

    mesh = plsc.VectorSubcoreMesh(core_axis_name="c", subcore_axis_name="s")

    @functools.partial(
        pl.kernel,
        mesh=mesh,
        compiler_params=pltpu.CompilerParams(use_tc_tiling_on_sc=False),
        out_type=jax.ShapeDtypeStruct((rows, emb), jnp.float32),
        scratch_types=[
            pltpu.VMEM((ng, _IDXW), jnp.int32),
            pltpu.VMEM((chunk, emb), jnp.float32),
            pltpu.SemaphoreType.DMA,
        ],
    )
    def k(idx_hbm, table_hbm, postile_hbm, out_hbm, idx_v, rows_v, sem):
        wid = lax.axis_index("s") * nc + lax.axis_index("c")
        base = wid * rpw

        def chunk_body(c, carry):
            r0 = pl.multiple_of(base + c * chunk, 8)
            irow = pl.multiple_of(base // _IDXW + c * ng, 8)
            pltpu.sync_copy(postile_hbm, rows_v)
            pltpu.sync_copy(idx_hbm.at[pl.ds(irow, ng)], idx_v)
            copies = [
                pltpu.async_copy(
                    table_hbm.at[idx_v.at[g]],
                    rows_v.at[pl.ds(g * _IDXW, _IDXW)],
                    sem,
                    add=True,
                )
                for g in range(ng)
            ]
            for cp in copies:
                cp.wait()
            pltpu.sync_copy(rows_v, out_hbm.at[pl.ds(r0, chunk)])
            return carry

        lax.fori_loop(0, nch, chunk_body, None)

    return k


def kernel(input_tokens, token_table, pos_table):
    b, s = input_tokens.shape
    emb = token_table.shape[1]
    rows = b * s
    idx = input_tokens.astype(jnp.int32).reshape(rows // _IDXW, _IDXW)
    postile = jnp.tile(pos_table[:s], (4, 1))
    out = _build_embed(rows, emb, s)(idx, token_table, postile)
    return out.reshape(b, s, emb)


# speedup vs baseline: 1.0564x; 1.0564x over previous
import functools

import jax
import jax.numpy as jnp
from jax import lax
from jax.experimental import pallas as pl
from jax.experimental.pallas import tpu as pltpu
from jax.experimental.pallas import tpu_sc as plsc

_LANES = 16
_IDXW = 100


@functools.lru_cache(maxsize=None)
def _build_embed(rows, emb, seq):
    info = plsc.get_sparse_core_info()
    nc, ns = info.num_cores, info.num_subcores
    nw = nc * ns
    assert rows % nw == 0
    rpw = rows // nw
    chunk = 4 * seq
    assert rpw % chunk == 0 and chunk % _IDXW == 0
    nch = rpw // chunk
    ng = chunk // _IDXW
    nvec = emb // _LANES
    assert emb % _LANES == 0
    nrep = chunk // seq

    mesh = plsc.VectorSubcoreMesh(core_axis_name="c", subcore_axis_name="s")

    @functools.partial(
        pl.kernel,
        mesh=mesh,
        compiler_params=pltpu.CompilerParams(use_tc_tiling_on_sc=False),
        out_type=jax.ShapeDtypeStruct((rows, emb), jnp.float32),
        scratch_types=[
            pltpu.VMEM((2 * ng, _IDXW), jnp.int32),
            pltpu.VMEM((2 * chunk, emb), jnp.float32),
            pltpu.VMEM((seq, emb), jnp.float32),
            pltpu.SemaphoreType.DMA,
            pltpu.SemaphoreType.DMA,
        ],
    )
    def k(idx_hbm, table_hbm, pos_hbm, out_hbm, idx_v, rows_v, pos_v, sem0, sem1):
        wid = lax.axis_index("s") * nc + lax.axis_index("c")
        base = wid * rpw
        sems = (sem0, sem1)
        pltpu.sync_copy(pos_hbm.at[pl.ds(0, seq)], pos_v)

        def start_chunk(c, p):
            irow = pl.multiple_of(base // _IDXW + c * ng, 8)
            pltpu.sync_copy(
                idx_hbm.at[pl.ds(irow, ng)], idx_v.at[pl.ds(p * ng, ng)]
            )
            return [
                pltpu.async_copy(
                    table_hbm.at[idx_v.at[p * ng + g]],
                    rows_v.at[pl.ds(p * chunk + g * _IDXW, _IDXW)],
                    sems[p],
                )
                for g in range(ng)
            ]

        pending = start_chunk(0, 0)
        for c in range(nch):
            p = c % 2
            for cp in pending:
                cp.wait()
            if c + 1 < nch:
                pending = start_chunk(c + 1, 1 - p)

            def add_body(s, carry):
                for e in range(nvec):
                    pv = pos_v[s, pl.ds(e * _LANES, _LANES)]
                    for q in range(nrep):
                        r = p * chunk + q * seq + s
                        rows_v[r, pl.ds(e * _LANES, _LANES)] = (
                            rows_v[r, pl.ds(e * _LANES, _LANES)] + pv
                        )
                return carry

            lax.fori_loop(0, seq, add_body, None)
            r0 = pl.multiple_of(base + c * chunk, 8)
            pltpu.sync_copy(
                rows_v.at[pl.ds(p * chunk, chunk)], out_hbm.at[pl.ds(r0, chunk)]
            )

    return k


def kernel(input_tokens, token_table, pos_table):
    b, s = input_tokens.shape
    emb = token_table.shape[1]
    rows = b * s
    idx = input_tokens.astype(jnp.int32).reshape(rows // _IDXW, _IDXW)
    out = _build_embed(rows, emb, s)(idx, token_table, pos_table)
    return out.reshape(b, s, emb)
